# baseline (speedup 1.0000x reference)
import functools

import jax
import jax.numpy as jnp
import numpy as np
from jax import lax
from jax.experimental import pallas as pl
from jax.experimental.pallas import tpu as pltpu
from jax.experimental.pallas import tpu_sc as plsc

_B = 128
_IMG = 3 * 224 * 224
_TOT = _B * _IMG
_NB_OP = 36
_PAD_OPS = 48
_NW = 32
_NG = _B // 16
_CHUNK = 18816
_NCHW = _TOT // (_NW * _CHUNK)
_GVECS = _CHUNK // _B
_BIG = np.float32(1e30)


def _op_tables():
    c = np.full(_PAD_OPS, -_BIG, np.float32)
    a2 = np.ones(_PAD_OPS, np.float32)
    b2 = np.zeros(_PAD_OPS, np.float32)
    lo = np.full(_PAD_OPS, -_BIG, np.float32)
    hi = np.full(_PAD_OPS, _BIG, np.float32)
    names = ["Identity", "Brightness", "Contrast", "Solarize"]
    for op in range(_NB_OP):
        name = names[op // 9]
        mag = (op % 9 + 1) / 10.0
        if name == "Brightness":
            b2[op] = np.float32(mag)
            lo[op], hi[op] = 0.0, 1.0
        elif name == "Contrast":
            a2[op] = np.float32(1.0 + mag)
            lo[op], hi[op] = 0.0, 1.0
        elif name == "Solarize":
            c[op] = np.float32(mag)
            a2[op] = -1.0
            b2[op] = 1.0
    return np.concatenate([c, a2, b2, lo, hi])


def _sc_body(g_hbm, u_hbm, tab_hbm, x_hbm, out_hbm,
             gbuf, ubuf, tbuf, pbuf, in0, in1, in2, ou0, ou1, ou2,
             si0, si1, si2, so0, so1, so2):
    cid = lax.axis_index("c")
    sid = lax.axis_index("s")
    wid = sid * 2 + cid
    base = wid * _NCHW * _CHUNK

    def prime_in(t, p, ins, sis):
        pltpu.async_copy(x_hbm.at[pl.ds(base + t * _CHUNK, _CHUNK)], ins[p], sis[p])

    prime_in(0, 0, (in0, in1, in2), (si0, si1, si2))
    prime_in(1, 1, (in0, in1, in2), (si0, si1, si2))
    prime_in(2, 2, (in0, in1, in2), (si0, si1, si2))

    pltpu.sync_copy(g_hbm, gbuf)
    pltpu.sync_copy(u_hbm, ubuf)
    pltpu.sync_copy(tab_hbm, tbuf)

    neg = jnp.full((16,), -_BIG, jnp.float32)
    zero_i = jnp.zeros((16,), jnp.int32)
    init = [neg] * _NG + [zero_i] * _NG

    def samp(j, st):
        mval, midx = st[:_NG], st[_NG:]
        nmval, nmidx = [], []
        for g in range(_NG):
            v = gbuf[pl.ds(j * _B + g * 16, 16)]
            better = v > mval[g]
            nmval.append(jnp.where(better, v, mval[g]))
            nmidx.append(jnp.where(better, j, midx[g]))
        return nmval + nmidx

    st = lax.fori_loop(0, _NB_OP, samp, init)
    midx = st[_NG:]

    one = jnp.float32(1.0)
    for g in range(_NG):
        uo = plsc.load_gather(ubuf, [midx[g]])
        coin = uo < jnp.float32(0.5)
        c_v = jnp.where(coin, plsc.load_gather(tbuf, [midx[g]]), -_BIG)
        a2_v = jnp.where(coin, plsc.load_gather(tbuf, [midx[g] + 48]), one)
        b2_v = jnp.where(coin, plsc.load_gather(tbuf, [midx[g] + 96]),
                         jnp.float32(0.0))
        lo_v = jnp.where(coin, plsc.load_gather(tbuf, [midx[g] + 144]), -_BIG)
        hi_v = jnp.where(coin, plsc.load_gather(tbuf, [midx[g] + 192]), _BIG)
        pbuf[pl.ds(g * 80, 16)] = c_v
        pbuf[pl.ds(g * 80 + 16, 16)] = a2_v
        pbuf[pl.ds(g * 80 + 32, 16)] = b2_v
        pbuf[pl.ds(g * 80 + 48, 16)] = lo_v
        pbuf[pl.ds(g * 80 + 64, 16)] = hi_v

    ins = (in0, in1, in2)
    ous = (ou0, ou1, ou2)
    sis = (si0, si1, si2)
    sos = (so0, so1, so2)

    def issue_in(t, p):
        pltpu.async_copy(x_hbm.at[pl.ds(base + t * _CHUNK, _CHUNK)], ins[p], sis[p])

    def issue_out(t, p):
        pltpu.async_copy(ous[p], out_hbm.at[pl.ds(base + t * _CHUNK, _CHUNK)], sos[p])

    def wait_in(p):
        pltpu.make_async_copy(x_hbm.at[pl.ds(0, _CHUNK)], ins[p], sis[p]).wait()

    def wait_out(p):
        pltpu.make_async_copy(ous[p], out_hbm.at[pl.ds(0, _CHUNK)], sos[p]).wait()

    def step(t, p):
        wait_in(p)

        @pl.when(t >= 3)
        def _():
            wait_out(p)

        inb, oub = ins[p], ous[p]
        for g in range(_NG):
            c_v = pbuf[pl.ds(g * 80, 16)]
            a2_v = pbuf[pl.ds(g * 80 + 16, 16)]
            b2_v = pbuf[pl.ds(g * 80 + 32, 16)]
            lo_v = pbuf[pl.ds(g * 80 + 48, 16)]
            hi_v = pbuf[pl.ds(g * 80 + 64, 16)]

            @plsc.parallel_loop(g * 16, g * 16 + _CHUNK, step=_B, unroll=7)
            def _(off):
                xv = inb[pl.ds(off, 16)]
                y = jnp.where(xv < c_v, xv, a2_v * xv + b2_v)
                oub[pl.ds(off, 16)] = jnp.minimum(jnp.maximum(y, lo_v), hi_v)

        @pl.when(t < _NCHW - 3)
        def _():
            issue_in(t + 3, p)

        issue_out(t, p)

    def body(h, _):
        for p in (0, 1, 2):
            step(3 * h + p, p)
        return 0

    lax.fori_loop(0, _NCHW // 3, body, 0)
    step(_NCHW - 2, (_NCHW - 2) % 3)
    step(_NCHW - 1, (_NCHW - 1) % 3)
    wait_out((_NCHW - 3) % 3)
    wait_out((_NCHW - 2) % 3)
    wait_out((_NCHW - 1) % 3)


@jax.jit
def _run(x_flat, g, u, tab):
    mesh = plsc.VectorSubcoreMesh(core_axis_name="c", subcore_axis_name="s")
    f = pl.kernel(
        _sc_body,
        out_type=jax.ShapeDtypeStruct((_TOT,), jnp.float32),
        mesh=mesh,
        compiler_params=pltpu.CompilerParams(needs_layout_passes=False),
        scratch_types=[
            pltpu.VMEM((_NB_OP * _B,), jnp.float32),
            pltpu.VMEM((_PAD_OPS,), jnp.float32),
            pltpu.VMEM((5 * _PAD_OPS,), jnp.float32),
            pltpu.VMEM((_NG * 80,), jnp.float32),
            pltpu.VMEM((_CHUNK,), jnp.float32),
            pltpu.VMEM((_CHUNK,), jnp.float32),
            pltpu.VMEM((_CHUNK,), jnp.float32),
            pltpu.VMEM((_CHUNK,), jnp.float32),
            pltpu.VMEM((_CHUNK,), jnp.float32),
            pltpu.VMEM((_CHUNK,), jnp.float32),
            pltpu.SemaphoreType.DMA,
            pltpu.SemaphoreType.DMA,
            pltpu.SemaphoreType.DMA,
            pltpu.SemaphoreType.DMA,
            pltpu.SemaphoreType.DMA,
            pltpu.SemaphoreType.DMA,
        ],
    )
    return f(g, u, tab, x_flat)


def kernel(x):
    key = jax.random.key(1)
    k = jax.random.fold_in(key, 0)
    g = jax.random.gumbel(jax.random.fold_in(k, 0), (_B, _NB_OP), jnp.float32)
    u = jax.random.uniform(jax.random.fold_in(k, 1), (_NB_OP,), jnp.float32)
    g_opmajor = g.T.reshape(-1)
    u48 = jnp.concatenate(
        [u, jnp.ones((_PAD_OPS - _NB_OP,), jnp.float32)])
    tab = jnp.asarray(_op_tables())
    x_t = jnp.transpose(x, (1, 2, 3, 0))
    out = _run(x_t.reshape(-1), g_opmajor, u48, tab)
    return jnp.transpose(out.reshape(3, 224, 224, _B), (3, 0, 1, 2))


# speedup vs baseline: 1.0072x; 1.0072x over previous
import functools

import jax
import jax.numpy as jnp
import numpy as np
from jax import lax
from jax.experimental import pallas as pl
from jax.experimental.pallas import tpu as pltpu
from jax.experimental.pallas import tpu_sc as plsc

_B = 128
_IMG = 3 * 224 * 224
_TOT = _B * _IMG
_NB_OP = 36
_PAD_OPS = 48
_NW = 32
_NG = _B // 16
_CHUNK = 28672
_NCHW = _TOT // (_NW * _CHUNK)
_GVECS = _CHUNK // _B
_BIG = np.float32(1e30)


def _op_tables():
    c = np.full(_PAD_OPS, -_BIG, np.float32)
    a2 = np.ones(_PAD_OPS, np.float32)
    b2 = np.zeros(_PAD_OPS, np.float32)
    lo = np.full(_PAD_OPS, -_BIG, np.float32)
    hi = np.full(_PAD_OPS, _BIG, np.float32)
    names = ["Identity", "Brightness", "Contrast", "Solarize"]
    for op in range(_NB_OP):
        name = names[op // 9]
        mag = (op % 9 + 1) / 10.0
        if name == "Brightness":
            b2[op] = np.float32(mag)
            lo[op], hi[op] = 0.0, 1.0
        elif name == "Contrast":
            a2[op] = np.float32(1.0 + mag)
            lo[op], hi[op] = 0.0, 1.0
        elif name == "Solarize":
            c[op] = np.float32(mag)
            a2[op] = -1.0
            b2[op] = 1.0
    return np.concatenate([c, a2, b2, lo, hi])


def _sc_body(g_hbm, u_hbm, tab_hbm, x_hbm, out_hbm,
             gbuf, ubuf, tbuf, pbuf, in0, in1, ou0, ou1,
             si0, si1, so0, so1):
    cid = lax.axis_index("c")
    sid = lax.axis_index("s")
    wid = sid * 2 + cid
    base = wid * _NCHW * _CHUNK

    def prime_in(t, p, ins, sis):
        pltpu.async_copy(x_hbm.at[pl.ds(base + t * _CHUNK, _CHUNK)], ins[p], sis[p])

    prime_in(0, 0, (in0, in1), (si0, si1))
    prime_in(1, 1, (in0, in1), (si0, si1))

    pltpu.sync_copy(g_hbm, gbuf)
    pltpu.sync_copy(u_hbm, ubuf)
    pltpu.sync_copy(tab_hbm, tbuf)

    neg = jnp.full((16,), -_BIG, jnp.float32)
    zero_i = jnp.zeros((16,), jnp.int32)
    init = [neg] * _NG + [zero_i] * _NG

    def samp(j, st):
        mval, midx = st[:_NG], st[_NG:]
        nmval, nmidx = [], []
        for g in range(_NG):
            v = gbuf[pl.ds(j * _B + g * 16, 16)]
            better = v > mval[g]
            nmval.append(jnp.where(better, v, mval[g]))
            nmidx.append(jnp.where(better, j, midx[g]))
        return nmval + nmidx

    st = lax.fori_loop(0, _NB_OP, samp, init)
    midx = st[_NG:]

    one = jnp.float32(1.0)
    for g in range(_NG):
        uo = plsc.load_gather(ubuf, [midx[g]])
        coin = uo < jnp.float32(0.5)
        c_v = jnp.where(coin, plsc.load_gather(tbuf, [midx[g]]), -_BIG)
        a2_v = jnp.where(coin, plsc.load_gather(tbuf, [midx[g] + 48]), one)
        b2_v = jnp.where(coin, plsc.load_gather(tbuf, [midx[g] + 96]),
                         jnp.float32(0.0))
        lo_v = jnp.where(coin, plsc.load_gather(tbuf, [midx[g] + 144]), -_BIG)
        hi_v = jnp.where(coin, plsc.load_gather(tbuf, [midx[g] + 192]), _BIG)
        pbuf[pl.ds(g * 80, 16)] = c_v
        pbuf[pl.ds(g * 80 + 16, 16)] = a2_v
        pbuf[pl.ds(g * 80 + 32, 16)] = b2_v
        pbuf[pl.ds(g * 80 + 48, 16)] = lo_v
        pbuf[pl.ds(g * 80 + 64, 16)] = hi_v

    ins = (in0, in1)
    ous = (ou0, ou1)
    sis = (si0, si1)
    sos = (so0, so1)

    def issue_in(t, p):
        pltpu.async_copy(x_hbm.at[pl.ds(base + t * _CHUNK, _CHUNK)], ins[p], sis[p])

    def issue_out(t, p):
        pltpu.async_copy(ous[p], out_hbm.at[pl.ds(base + t * _CHUNK, _CHUNK)], sos[p])

    def wait_in(p):
        pltpu.make_async_copy(x_hbm.at[pl.ds(0, _CHUNK)], ins[p], sis[p]).wait()

    def wait_out(p):
        pltpu.make_async_copy(ous[p], out_hbm.at[pl.ds(0, _CHUNK)], sos[p]).wait()

    def step(t, p):
        wait_in(p)

        @pl.when(t >= 2)
        def _():
            wait_out(p)

        inb, oub = ins[p], ous[p]
        for g in range(_NG):
            c_v = pbuf[pl.ds(g * 80, 16)]
            a2_v = pbuf[pl.ds(g * 80 + 16, 16)]
            b2_v = pbuf[pl.ds(g * 80 + 32, 16)]
            lo_v = pbuf[pl.ds(g * 80 + 48, 16)]
            hi_v = pbuf[pl.ds(g * 80 + 64, 16)]

            @plsc.parallel_loop(g * 16, g * 16 + _CHUNK, step=_B, unroll=7)
            def _(off):
                xv = inb[pl.ds(off, 16)]
                y = jnp.where(xv < c_v, xv, a2_v * xv + b2_v)
                oub[pl.ds(off, 16)] = jnp.minimum(jnp.maximum(y, lo_v), hi_v)

        @pl.when(t < _NCHW - 2)
        def _():
            issue_in(t + 2, p)

        issue_out(t, p)

    def body(h, _):
        for p in (0, 1):
            step(2 * h + p, p)
        return 0

    lax.fori_loop(0, _NCHW // 2, body, 0)
    step(_NCHW - 1, (_NCHW - 1) % 2)
    wait_out((_NCHW - 2) % 2)
    wait_out((_NCHW - 1) % 2)


@jax.jit
def _run(x_flat, g, u, tab):
    mesh = plsc.VectorSubcoreMesh(core_axis_name="c", subcore_axis_name="s")
    f = pl.kernel(
        _sc_body,
        out_type=jax.ShapeDtypeStruct((_TOT,), jnp.float32),
        mesh=mesh,
        compiler_params=pltpu.CompilerParams(needs_layout_passes=False),
        scratch_types=[
            pltpu.VMEM((_NB_OP * _B,), jnp.float32),
            pltpu.VMEM((_PAD_OPS,), jnp.float32),
            pltpu.VMEM((5 * _PAD_OPS,), jnp.float32),
            pltpu.VMEM((_NG * 80,), jnp.float32),
            pltpu.VMEM((_CHUNK,), jnp.float32),
            pltpu.VMEM((_CHUNK,), jnp.float32),
            pltpu.VMEM((_CHUNK,), jnp.float32),
            pltpu.VMEM((_CHUNK,), jnp.float32),
            pltpu.SemaphoreType.DMA,
            pltpu.SemaphoreType.DMA,
            pltpu.SemaphoreType.DMA,
            pltpu.SemaphoreType.DMA,
        ],
    )
    return f(g, u, tab, x_flat)


def kernel(x):
    key = jax.random.key(1)
    k = jax.random.fold_in(key, 0)
    g = jax.random.gumbel(jax.random.fold_in(k, 0), (_B, _NB_OP), jnp.float32)
    u = jax.random.uniform(jax.random.fold_in(k, 1), (_NB_OP,), jnp.float32)
    g_opmajor = g.T.reshape(-1)
    u48 = jnp.concatenate(
        [u, jnp.ones((_PAD_OPS - _NB_OP,), jnp.float32)])
    tab = jnp.asarray(_op_tables())
    x_t = jnp.transpose(x, (1, 2, 3, 0))
    out = _run(x_t.reshape(-1), g_opmajor, u48, tab)
    return jnp.transpose(out.reshape(3, 224, 224, _B), (3, 0, 1, 2))


# speedup vs baseline: 1.0180x; 1.0107x over previous
import jax
import jax.numpy as jnp
import numpy as np
from jax import lax
from jax.experimental import pallas as pl
from jax.experimental.pallas import tpu as pltpu
from jax.experimental.pallas import tpu_sc as plsc

_B = 128
_IMG = 3 * 224 * 224
_TOT = _B * _IMG
_NB_OP = 36
_PAD_OPS = 48
_NW = 32
_NG = _B // 16
_CHUNK = 25088
_NCHW = _TOT // (_NW * _CHUNK)
_GVECS = _CHUNK // _B
_BIG = np.float32(1e30)


def _op_tables():
    c = np.full(_PAD_OPS, -_BIG, np.float32)
    a2 = np.ones(_PAD_OPS, np.float32)
    b2 = np.zeros(_PAD_OPS, np.float32)
    lo = np.full(_PAD_OPS, -_BIG, np.float32)
    hi = np.full(_PAD_OPS, _BIG, np.float32)
    names = ["Identity", "Brightness", "Contrast", "Solarize"]
    for op in range(_NB_OP):
        name = names[op // 9]
        mag = (op % 9 + 1) / 10.0
        if name == "Brightness":
            b2[op] = np.float32(mag)
            lo[op], hi[op] = 0.0, 1.0
        elif name == "Contrast":
            a2[op] = np.float32(1.0 + mag)
            lo[op], hi[op] = 0.0, 1.0
        elif name == "Solarize":
            c[op] = np.float32(mag)
            a2[op] = -1.0
            b2[op] = 1.0
    return np.concatenate([c, a2, b2, lo, hi])


def _sc_body(g_hbm, u_hbm, tab_hbm, x_hbm, out_hbm,
             gbuf, ubuf, tbuf, pbuf, in0, in1, ou0, ou1,
             si0, si1, so0, so1):
    cid = lax.axis_index("c")
    sid = lax.axis_index("s")
    wid = sid * 2 + cid
    base = wid * _NCHW * _CHUNK

    def prime_in(t, p, ins, sis):
        pltpu.async_copy(x_hbm.at[pl.ds(base + t * _CHUNK, _CHUNK)], ins[p], sis[p])

    prime_in(0, 0, (in0, in1), (si0, si1))
    prime_in(1, 1, (in0, in1), (si0, si1))

    pltpu.sync_copy(g_hbm, gbuf)
    pltpu.sync_copy(u_hbm, ubuf)
    pltpu.sync_copy(tab_hbm, tbuf)

    neg = jnp.full((16,), -_BIG, jnp.float32)
    zero_i = jnp.zeros((16,), jnp.int32)
    init = [neg] * _NG + [zero_i] * _NG

    def samp(j, st):
        mval, midx = st[:_NG], st[_NG:]
        nmval, nmidx = [], []
        for g in range(_NG):
            v = gbuf[pl.ds(j * _B + g * 16, 16)]
            better = v > mval[g]
            nmval.append(jnp.where(better, v, mval[g]))
            nmidx.append(jnp.where(better, j, midx[g]))
        return nmval + nmidx

    st = lax.fori_loop(0, _NB_OP, samp, init)
    midx = st[_NG:]

    one = jnp.float32(1.0)
    for g in range(_NG):
        uo = plsc.load_gather(ubuf, [midx[g]])
        coin = uo < jnp.float32(0.5)
        c_v = jnp.where(coin, plsc.load_gather(tbuf, [midx[g]]), -_BIG)
        a2_v = jnp.where(coin, plsc.load_gather(tbuf, [midx[g] + 48]), one)
        b2_v = jnp.where(coin, plsc.load_gather(tbuf, [midx[g] + 96]),
                         jnp.float32(0.0))
        lo_v = jnp.where(coin, plsc.load_gather(tbuf, [midx[g] + 144]), -_BIG)
        hi_v = jnp.where(coin, plsc.load_gather(tbuf, [midx[g] + 192]), _BIG)
        pbuf[pl.ds(g * 80, 16)] = c_v
        pbuf[pl.ds(g * 80 + 16, 16)] = a2_v
        pbuf[pl.ds(g * 80 + 32, 16)] = b2_v
        pbuf[pl.ds(g * 80 + 48, 16)] = lo_v
        pbuf[pl.ds(g * 80 + 64, 16)] = hi_v

    ins = (in0, in1)
    ous = (ou0, ou1)
    sis = (si0, si1)
    sos = (so0, so1)

    def issue_in(t, p):
        pltpu.async_copy(x_hbm.at[pl.ds(base + t * _CHUNK, _CHUNK)], ins[p], sis[p])

    def issue_out(t, p):
        pltpu.async_copy(ous[p], out_hbm.at[pl.ds(base + t * _CHUNK, _CHUNK)], sos[p])

    def wait_in(p):
        pltpu.make_async_copy(x_hbm.at[pl.ds(0, _CHUNK)], ins[p], sis[p]).wait()

    def wait_out(p):
        pltpu.make_async_copy(ous[p], out_hbm.at[pl.ds(0, _CHUNK)], sos[p]).wait()

    def body(h, _):
        for p in (0, 1):
            t = 2 * h + p
            wait_in(p)

            @pl.when(t >= 2)
            def _():
                wait_out(p)

            inb, oub = ins[p], ous[p]
            for g in range(_NG):
                c_v = pbuf[pl.ds(g * 80, 16)]
                a2_v = pbuf[pl.ds(g * 80 + 16, 16)]
                b2_v = pbuf[pl.ds(g * 80 + 32, 16)]
                lo_v = pbuf[pl.ds(g * 80 + 48, 16)]
                hi_v = pbuf[pl.ds(g * 80 + 64, 16)]

                @plsc.parallel_loop(g * 16, g * 16 + _CHUNK, step=_B, unroll=7)
                def _(off):
                    xv = inb[pl.ds(off, 16)]
                    y = jnp.where(xv < c_v, xv, a2_v * xv + b2_v)
                    oub[pl.ds(off, 16)] = jnp.minimum(jnp.maximum(y, lo_v), hi_v)

            @pl.when(t < _NCHW - 2)
            def _():
                issue_in(t + 2, p)

            issue_out(t, p)
        return 0

    lax.fori_loop(0, _NCHW // 2, body, 0)
    wait_out(0)
    wait_out(1)


@jax.jit
def _run(x_flat, g, u, tab):
    mesh = plsc.VectorSubcoreMesh(core_axis_name="c", subcore_axis_name="s")
    f = pl.kernel(
        _sc_body,
        out_type=jax.ShapeDtypeStruct((_TOT,), jnp.float32),
        mesh=mesh,
        compiler_params=pltpu.CompilerParams(needs_layout_passes=False),
        scratch_types=[
            pltpu.VMEM((_NB_OP * _B,), jnp.float32),
            pltpu.VMEM((_PAD_OPS,), jnp.float32),
            pltpu.VMEM((5 * _PAD_OPS,), jnp.float32),
            pltpu.VMEM((_NG * 80,), jnp.float32),
            pltpu.VMEM((_CHUNK,), jnp.float32),
            pltpu.VMEM((_CHUNK,), jnp.float32),
            pltpu.VMEM((_CHUNK,), jnp.float32),
            pltpu.VMEM((_CHUNK,), jnp.float32),
            pltpu.SemaphoreType.DMA,
            pltpu.SemaphoreType.DMA,
            pltpu.SemaphoreType.DMA,
            pltpu.SemaphoreType.DMA,
        ],
    )
    return f(g, u, tab, x_flat)


def kernel(x):
    key = jax.random.key(1)
    k = jax.random.fold_in(key, 0)
    g = jax.random.gumbel(jax.random.fold_in(k, 0), (_B, _NB_OP), jnp.float32)
    u = jax.random.uniform(jax.random.fold_in(k, 1), (_NB_OP,), jnp.float32)
    g_opmajor = g.T.reshape(-1)
    u48 = jnp.concatenate(
        [u, jnp.ones((_PAD_OPS - _NB_OP,), jnp.float32)])
    tab = jnp.asarray(_op_tables())
    x_t = jnp.transpose(x, (1, 2, 3, 0))
    out = _run(x_t.reshape(-1), g_opmajor, u48, tab)
    return jnp.transpose(out.reshape(3, 224, 224, _B), (3, 0, 1, 2))
